# Initial kernel scaffold; baseline (speedup 1.0000x reference)
import jax
import jax.numpy as jnp
from jax.experimental import pallas as pl


def kernel(x, edge_index_0, edge_index_1, gat_W_0, gat_al_0, gat_ar_0, gat_b_0, gin_W1_0, gin_b1_0, gin_g_0, gin_be_0, gin_W2_0, gin_b2_0, gat_W_1, gat_al_1, gat_ar_1, gat_b_1, gin_W1_1, gin_b1_1, gin_g_1, gin_be_1, gin_W2_1, gin_b2_1):
    raise NotImplementedError("write your pallas kernel here")


# speedup vs baseline: 22.6272x; 22.6272x over previous
import functools

import jax
import jax.numpy as jnp
from jax import lax
from jax.experimental import pallas as pl
from jax.experimental.pallas import tpu as pltpu
from jax.experimental.pallas import tpu_sc as plsc

N = 10000
E = 320000
D = 128
H = 4
Fh = 32

NC = 2
NS = 16
NW = NC * NS
EPW = E // NW
EB = 80
NBLK = EPW // EB
RPS = 624
RLAST = N - 15 * RPS

NB = 10
BN = N // NB
EBT = 2000
NEB = E // EBT

f32 = jnp.float32


def _gmask():
    rr = lax.broadcasted_iota(jnp.int32, (D, 16), 0)
    cc = lax.broadcasted_iota(jnp.int32, (D, 16), 1)
    return ((rr // Fh) == cc).astype(f32)


def _pmask():
    rr = lax.broadcasted_iota(jnp.int32, (16, D), 0)
    cc = lax.broadcasted_iota(jnp.int32, (16, D), 1)
    return (((cc // Fh) == rr) & (rr < H)).astype(f32)


def _prep_body(x_ref, w0, al0, ar0, w1, al1, ar1,
               feat0, bmA0, bmB0, feat1, bmA1, bmB1):
    x = x_ref[...]
    G = _gmask()

    def one(w, alf, arf, featref, bmAref, bmBref):
        feat = jnp.dot(x, w[...], preferred_element_type=f32)
        featref[...] = feat
        el16 = jnp.dot(feat * alf[...], G, preferred_element_type=f32)
        er16 = jnp.dot(feat * arf[...], G, preferred_element_type=f32)
        bmAref[...] = jnp.max(el16, axis=0).reshape(1, 1, 16)
        bmBref[...] = jnp.max(er16, axis=0).reshape(1, 1, 16)

    one(w0, al0, ar0, feat0, bmA0, bmB0)
    one(w1, al1, ar1, feat1, bmA1, bmB1)


def _prep(x, W0, al0, ar0, W1, al1, ar1):
    full = lambda shp: pl.BlockSpec(shp, lambda i: tuple(0 for _ in shp))
    outs = (jax.ShapeDtypeStruct((N, D), f32),
            jax.ShapeDtypeStruct((NB, 1, 16), f32),
            jax.ShapeDtypeStruct((NB, 1, 16), f32))
    return pl.pallas_call(
        _prep_body,
        grid=(NB,),
        in_specs=[pl.BlockSpec((BN, D), lambda i: (i, 0)),
                  full((D, D)), full((1, D)), full((1, D)),
                  full((D, D)), full((1, D)), full((1, D))],
        out_specs=(pl.BlockSpec((BN, D), lambda i: (i, 0)),
                   pl.BlockSpec((1, 1, 16), lambda i: (i, 0, 0)),
                   pl.BlockSpec((1, 1, 16), lambda i: (i, 0, 0))) * 2,
        out_shape=outs * 2,
    )(x, W0, al0, ar0, W1, al1, ar1)


def _edge_body(FS_ref, FD_ref, bmA_ref, bmB_ref, alf_ref, arf_ref,
               msg_ref, ee_ref):
    FS = FS_ref[...]
    G = _gmask()
    els = jnp.dot(FS * alf_ref[...], G, preferred_element_type=f32)
    erd = jnp.dot(FD_ref[...] * arf_ref[...], G, preferred_element_type=f32)
    s = els + erd
    s = jnp.where(s >= 0, s, 0.2 * s)
    c = jnp.max(bmA_ref[...][:, 0, :], axis=0) + jnp.max(bmB_ref[...][:, 0, :], axis=0)
    bound = jnp.where(c >= 0, c, 0.2 * c)
    ee = jnp.exp(s - bound)
    P = _pmask()
    eex = jnp.dot(ee, P, preferred_element_type=f32)
    msg_ref[...] = FS * eex
    rr = lax.broadcasted_iota(jnp.int32, (16, D), 0)
    cc = lax.broadcasted_iota(jnp.int32, (16, D), 1)
    eye = ((rr == cc) & (rr < H)).astype(f32)
    ee_ref[...] = jnp.dot(ee, eye, preferred_element_type=f32)


def _edge(FS, FD, bmA, bmB, alf, arf):
    bmspec = pl.BlockSpec((NB, 1, 16), lambda i: (0, 0, 0))
    vec = pl.BlockSpec((1, D), lambda i: (0, 0))
    return pl.pallas_call(
        _edge_body,
        grid=(NEB,),
        in_specs=[pl.BlockSpec((EBT, D), lambda i: (i, 0)),
                  pl.BlockSpec((EBT, D), lambda i: (i, 0)),
                  bmspec, bmspec, vec, vec],
        out_specs=(pl.BlockSpec((EBT, D), lambda i: (i, 0)),
                   pl.BlockSpec((EBT, D), lambda i: (i, 0))),
        out_shape=(jax.ShapeDtypeStruct((E, D), f32),
                   jax.ShapeDtypeStruct((E, D), f32)),
    )(FS, FD, bmA, bmB, alf, arf)


def _norm_body(p128_ref, pden_ref, b_ref, h_ref):
    acc = p128_ref[0] + p128_ref[1]
    den = pden_ref[0] + pden_ref[1]
    rr = lax.broadcasted_iota(jnp.int32, (D, D), 0)
    cc = lax.broadcasted_iota(jnp.int32, (D, D), 1)
    PD = (((cc // Fh) == rr) & (rr < H)).astype(f32)
    denx = jnp.dot(den, PD, preferred_element_type=f32)
    out = acc / jnp.where(denx > 0, denx, 1.0) + b_ref[...]
    h_ref[...] = jnp.maximum(out, 0.0)


def _norm(p128, pden, b):
    return pl.pallas_call(
        _norm_body,
        grid=(NB,),
        in_specs=[pl.BlockSpec((NC, BN, D), lambda i: (0, i, 0)),
                  pl.BlockSpec((NC, BN, D), lambda i: (0, i, 0)),
                  pl.BlockSpec((1, D), lambda i: (0, 0))],
        out_specs=pl.BlockSpec((BN, D), lambda i: (i, 0)),
        out_shape=jax.ShapeDtypeStruct((N, D), f32),
    )(p128, pden, b)


def _gin1_body(h_ref, ph_ref, x_ref, px_ref, w1h_ref, w1x_ref, b1_ref,
               z_ref, bs_ref, bq_ref):
    hh = h_ref[...] + ph_ref[0] + ph_ref[1]
    xx = x_ref[...] + px_ref[0] + px_ref[1]
    z = (jnp.dot(hh, w1h_ref[...], preferred_element_type=f32)
         + jnp.dot(xx, w1x_ref[...], preferred_element_type=f32) + b1_ref[...])
    z_ref[...] = z
    bs_ref[...] = jnp.sum(z, axis=0).reshape(1, 1, D)
    bq_ref[...] = jnp.sum(z * z, axis=0).reshape(1, 1, D)


def _gin1(h, ph, x, px, W1h, W1x, b1):
    full = lambda shp: pl.BlockSpec(shp, lambda i: tuple(0 for _ in shp))
    return pl.pallas_call(
        _gin1_body,
        grid=(NB,),
        in_specs=[pl.BlockSpec((BN, D), lambda i: (i, 0)),
                  pl.BlockSpec((NC, BN, D), lambda i: (0, i, 0)),
                  pl.BlockSpec((BN, D), lambda i: (i, 0)),
                  pl.BlockSpec((NC, BN, D), lambda i: (0, i, 0)),
                  full((D, D)), full((D, D)), full((1, D))],
        out_specs=(pl.BlockSpec((BN, D), lambda i: (i, 0)),
                   pl.BlockSpec((1, 1, D), lambda i: (i, 0, 0)),
                   pl.BlockSpec((1, 1, D), lambda i: (i, 0, 0))),
        out_shape=(jax.ShapeDtypeStruct((N, D), f32),
                   jax.ShapeDtypeStruct((NB, 1, D), f32),
                   jax.ShapeDtypeStruct((NB, 1, D), f32)),
    )(h, ph, x, px, W1h, W1x, b1)


def _gin2_body(z_ref, bs_ref, bq_ref, g_ref, be_ref, w2_ref, b2_ref, o_ref):
    mu = jnp.sum(bs_ref[...][:, 0, :], axis=0) * (1.0 / N)
    msq = jnp.sum(bq_ref[...][:, 0, :], axis=0) * (1.0 / N)
    var = msq - mu * mu
    inv = lax.rsqrt(var + 1e-5)
    z = (z_ref[...] - mu) * (inv * g_ref[...]) + be_ref[...]
    z = jnp.maximum(z, 0.0)
    o = jnp.dot(z, w2_ref[...], preferred_element_type=f32) + b2_ref[...]
    o_ref[...] = jnp.maximum(o, 0.0)


def _gin2(z, bs, bq, g, be, W2, b2):
    full = lambda shp: pl.BlockSpec(shp, lambda i: tuple(0 for _ in shp))
    return pl.pallas_call(
        _gin2_body,
        grid=(NB,),
        in_specs=[pl.BlockSpec((BN, D), lambda i: (i, 0)),
                  full((NB, 1, D)), full((NB, 1, D)),
                  full((1, D)), full((1, D)), full((D, D)), full((1, D))],
        out_specs=pl.BlockSpec((BN, D), lambda i: (i, 0)),
        out_shape=jax.ShapeDtypeStruct((N, D), f32),
    )(z, bs, bq, g, be, W2, b2)


def _sc_mesh():
    return plsc.VectorSubcoreMesh(core_axis_name="c", subcore_axis_name="s",
                                  num_cores=NC, num_subcores=NS)


def _gat_gather(feat, src3, dst3):
    @functools.partial(
        pl.kernel,
        out_type=(jax.ShapeDtypeStruct((E, D), f32),
                  jax.ShapeDtypeStruct((E, D), f32)),
        mesh=_sc_mesh(),
        scratch_types=[pltpu.VMEM((EB,), jnp.int32), pltpu.VMEM((EB,), jnp.int32),
                       pltpu.VMEM((EB, D), f32), pltpu.VMEM((EB, D), f32),
                       pltpu.SemaphoreType.DMA, pltpu.SemaphoreType.DMA],
    )
    def k(feat_hbm, src_hbm, dst_hbm, fs_hbm, fd_hbm,
          src_v, dst_v, rs_v, rd_v, sem1, sem2):
        c = lax.axis_index("c")
        s = lax.axis_index("s")
        wid = s * NC + c

        def body(j, carry):
            pltpu.sync_copy(src_hbm.at[wid, j], src_v)
            pltpu.sync_copy(dst_hbm.at[wid, j], dst_v)
            cp1 = pltpu.async_copy(feat_hbm.at[src_v], rs_v, sem1)
            cp2 = pltpu.async_copy(feat_hbm.at[dst_v], rd_v, sem2)
            cp1.wait()
            cp2.wait()
            base = (wid * NBLK + j) * EB
            pltpu.sync_copy(rs_v, fs_hbm.at[pl.ds(base, EB)])
            pltpu.sync_copy(rd_v, fd_hbm.at[pl.ds(base, EB)])
            return carry

        lax.fori_loop(0, NBLK, body, 0)

    return k(feat, src3, dst3)


def _scatter_rows(rows, dst3, zer128):
    @functools.partial(
        pl.kernel,
        out_type=jax.ShapeDtypeStruct((NC, N, D), f32),
        mesh=_sc_mesh(),
        scratch_types=[pltpu.VMEM((EB,), jnp.int32), pltpu.VMEM((EB, D), f32),
                       pltpu.VMEM_SHARED((N, D), f32)],
    )
    def k(rows_hbm, dst_hbm, z128_hbm, p_hbm, dst_v, m_v, acc_sh):
        c = lax.axis_index("c")
        s = lax.axis_index("s")
        wid = s * NC + c
        r0 = s * RPS

        @pl.when(s < NS - 1)
        def _():
            pltpu.sync_copy(z128_hbm.at[pl.ds(0, RPS)], acc_sh.at[pl.ds(r0, RPS)])

        @pl.when(s == NS - 1)
        def _():
            pltpu.sync_copy(z128_hbm, acc_sh.at[pl.ds(r0, RLAST)])

        plsc.subcore_barrier()

        def body(j, carry):
            pltpu.sync_copy(dst_hbm.at[wid, j], dst_v)
            base = (wid * NBLK + j) * EB
            pltpu.sync_copy(rows_hbm.at[pl.ds(base, EB)], m_v)
            pltpu.sync_copy(m_v, acc_sh.at[dst_v], add=True)
            return carry

        lax.fori_loop(0, NBLK, body, 0)
        plsc.subcore_barrier()

        @pl.when(s < NS - 1)
        def _():
            pltpu.sync_copy(acc_sh.at[pl.ds(r0, RPS)], p_hbm.at[c, pl.ds(r0, RPS)])

        @pl.when(s == NS - 1)
        def _():
            pltpu.sync_copy(acc_sh.at[pl.ds(r0, RLAST)], p_hbm.at[c, pl.ds(r0, RLAST)])

    return k(rows, dst3, zer128)


def _gsa(table, src3, dst3, zer128):
    @functools.partial(
        pl.kernel,
        out_type=jax.ShapeDtypeStruct((NC, N, D), f32),
        mesh=_sc_mesh(),
        scratch_types=[pltpu.VMEM((EB,), jnp.int32), pltpu.VMEM((EB,), jnp.int32),
                       pltpu.VMEM((EB, D), f32), pltpu.VMEM_SHARED((N, D), f32),
                       pltpu.SemaphoreType.DMA],
    )
    def k(tab_hbm, src_hbm, dst_hbm, z128_hbm, p_hbm,
          src_v, dst_v, rows_v, acc_sh, sem):
        c = lax.axis_index("c")
        s = lax.axis_index("s")
        wid = s * NC + c
        r0 = s * RPS

        @pl.when(s < NS - 1)
        def _():
            pltpu.sync_copy(z128_hbm.at[pl.ds(0, RPS)], acc_sh.at[pl.ds(r0, RPS)])

        @pl.when(s == NS - 1)
        def _():
            pltpu.sync_copy(z128_hbm, acc_sh.at[pl.ds(r0, RLAST)])

        plsc.subcore_barrier()

        def body(j, carry):
            pltpu.sync_copy(src_hbm.at[wid, j], src_v)
            pltpu.sync_copy(dst_hbm.at[wid, j], dst_v)
            pltpu.async_copy(tab_hbm.at[src_v], rows_v, sem).wait()
            pltpu.sync_copy(rows_v, acc_sh.at[dst_v], add=True)
            return carry

        lax.fori_loop(0, NBLK, body, 0)
        plsc.subcore_barrier()

        @pl.when(s < NS - 1)
        def _():
            pltpu.sync_copy(acc_sh.at[pl.ds(r0, RPS)], p_hbm.at[c, pl.ds(r0, RPS)])

        @pl.when(s == NS - 1)
        def _():
            pltpu.sync_copy(acc_sh.at[pl.ds(r0, RLAST)], p_hbm.at[c, pl.ds(r0, RLAST)])

    return k(table, src3, dst3, zer128)


def kernel(x, edge_index_0, edge_index_1,
           gat_W_0, gat_al_0, gat_ar_0, gat_b_0,
           gin_W1_0, gin_b1_0, gin_g_0, gin_be_0, gin_W2_0, gin_b2_0,
           gat_W_1, gat_al_1, gat_ar_1, gat_b_1,
           gin_W1_1, gin_b1_1, gin_g_1, gin_be_1, gin_W2_1, gin_b2_1):
    al0 = gat_al_0.reshape(1, D)
    ar0 = gat_ar_0.reshape(1, D)
    al1 = gat_al_1.reshape(1, D)
    ar1 = gat_ar_1.reshape(1, D)
    zer128 = jnp.zeros((RLAST, D), f32)

    feat0, bmA0, bmB0, feat1, bmA1, bmB1 = _prep(
        x, gat_W_0, al0, ar0, gat_W_1, al1, ar1)

    cfgs = [
        (edge_index_0, feat0, bmA0, bmB0, al0, ar0, gat_b_0,
         gin_W1_0, gin_b1_0, gin_g_0, gin_be_0, gin_W2_0, gin_b2_0),
        (edge_index_1, feat1, bmA1, bmB1, al1, ar1, gat_b_1,
         gin_W1_1, gin_b1_1, gin_g_1, gin_be_1, gin_W2_1, gin_b2_1),
    ]
    outs = []
    for (ei, feat, bmA, bmB, alf, arf, gb, W1, b1, g, be, W2, b2) in cfgs:
        src3 = ei[0].reshape(NW, NBLK, EB)
        dst3 = ei[1].reshape(NW, NBLK, EB)
        aggx_p = _gsa(x, src3, dst3, zer128)
        FS, FD = _gat_gather(feat, src3, dst3)
        msg, ee128 = _edge(FS, FD, bmA, bmB, alf, arf)
        p128 = _scatter_rows(msg, dst3, zer128)
        pden = _scatter_rows(ee128, dst3, zer128)
        h = _norm(p128, pden, gb.reshape(1, D))
        aggh_p = _gsa(h, src3, dst3, zer128)
        z, bs, bq = _gin1(h, aggh_p, x, aggx_p, W1[:D], W1[D:],
                          b1.reshape(1, D))
        o = _gin2(z, bs, bq, g.reshape(1, D), be.reshape(1, D),
                  W2, b2.reshape(1, D))
        outs.append(o)
    return jnp.concatenate(outs, axis=1)


# speedup vs baseline: 24.5384x; 1.0845x over previous
import functools

import jax
import jax.numpy as jnp
from jax import lax
from jax.experimental import pallas as pl
from jax.experimental.pallas import tpu as pltpu
from jax.experimental.pallas import tpu_sc as plsc

N = 10000
E = 320000
D = 128
H = 4
Fh = 32

NC = 2
NS = 16
NW = NC * NS
EPW = E // NW
EB = 80
NBLK = EPW // EB
RPS = 624
RLAST = N - 15 * RPS
DR = N // 8
NA = 11264
RPA = 704
RLA = NA - 15 * RPA

NB = 10
BN = N // NB

f32 = jnp.float32


def _gmask():
    rr = lax.broadcasted_iota(jnp.int32, (D, 16), 0)
    cc = lax.broadcasted_iota(jnp.int32, (D, 16), 1)
    return ((rr // Fh) == cc).astype(f32)


def _gmask128():
    rr = lax.broadcasted_iota(jnp.int32, (D, D), 0)
    cc = lax.broadcasted_iota(jnp.int32, (D, D), 1)
    return (((rr // Fh) == cc) & (cc < 16)).astype(f32)


def _pmask():
    rr = lax.broadcasted_iota(jnp.int32, (16, D), 0)
    cc = lax.broadcasted_iota(jnp.int32, (16, D), 1)
    return (((cc // Fh) == rr) & (rr < H)).astype(f32)


def _prep_body(x_ref, w0, al0, ar0, w1, al1, ar1,
               feat0, el0, er0, bmA0, bmB0,
               feat1, el1, er1, bmA1, bmB1):
    x = x_ref[...]
    G = _gmask()

    G128 = _gmask128()

    def one(w, alf, arf, featref, elref, erref, bmAref, bmBref):
        feat = jnp.dot(x, w[...], preferred_element_type=f32)
        featref[...] = feat
        el16 = jnp.dot(feat * alf[...], G, preferred_element_type=f32)
        er16 = jnp.dot(feat * arf[...], G, preferred_element_type=f32)
        elref[...] = jnp.dot(feat * alf[...], G128, preferred_element_type=f32)
        erref[...] = jnp.dot(feat * arf[...], G128, preferred_element_type=f32)
        bmAref[...] = jnp.max(el16, axis=0).reshape(1, 1, 16)
        bmBref[...] = jnp.max(er16, axis=0).reshape(1, 1, 16)

    one(w0, al0, ar0, feat0, el0, er0, bmA0, bmB0)
    one(w1, al1, ar1, feat1, el1, er1, bmA1, bmB1)


def _prep(x, W0, al0, ar0, W1, al1, ar1):
    full = lambda shp: pl.BlockSpec(shp, lambda i: tuple(0 for _ in shp))
    outs = (jax.ShapeDtypeStruct((N, D), f32),
            jax.ShapeDtypeStruct((N, D), f32),
            jax.ShapeDtypeStruct((N, D), f32),
            jax.ShapeDtypeStruct((NB, 1, 16), f32),
            jax.ShapeDtypeStruct((NB, 1, 16), f32))
    return pl.pallas_call(
        _prep_body,
        grid=(NB,),
        in_specs=[pl.BlockSpec((BN, D), lambda i: (i, 0)),
                  full((D, D)), full((1, D)), full((1, D)),
                  full((D, D)), full((1, D)), full((1, D))],
        out_specs=(pl.BlockSpec((BN, D), lambda i: (i, 0)),
                   pl.BlockSpec((BN, D), lambda i: (i, 0)),
                   pl.BlockSpec((BN, D), lambda i: (i, 0)),
                   pl.BlockSpec((1, 1, 16), lambda i: (i, 0, 0)),
                   pl.BlockSpec((1, 1, 16), lambda i: (i, 0, 0))) * 2,
        out_shape=outs * 2,
    )(x, W0, al0, ar0, W1, al1, ar1)


def _norm_body(p128_ref, pden_ref, b_ref, h_ref):
    i = pl.program_id(0)
    acc = p128_ref[0] + p128_ref[1]
    denp = pden_ref[0] + pden_ref[1]
    nn = lax.broadcasted_iota(jnp.int32, (BN, DR), 0)
    rc = lax.broadcasted_iota(jnp.int32, (BN, DR), 1)
    P = (rc == i * (BN // 8) + nn // 8).astype(f32)
    tmp = jnp.dot(P, denp, preferred_element_type=f32)
    ll = lax.broadcasted_iota(jnp.int32, (D, D), 0)
    dd = lax.broadcasted_iota(jnp.int32, (D, D), 1)
    nv = lax.broadcasted_iota(jnp.int32, (BN, 1), 0)
    denx = jnp.zeros_like(acc)
    for g in range(8):
        Mg = (ll == g * 16 + dd // Fh).astype(f32)
        mg = ((nv % 8) == g).astype(f32)
        denx = denx + mg * jnp.dot(tmp, Mg, preferred_element_type=f32)
    out = acc / jnp.where(denx > 0, denx, 1.0) + b_ref[...]
    h_ref[...] = jnp.maximum(out, 0.0)


def _norm(p128, denp, b):
    return pl.pallas_call(
        _norm_body,
        grid=(NB,),
        in_specs=[pl.BlockSpec((NC, BN, D), lambda i: (0, i, 0)),
                  pl.BlockSpec((NC, DR, D), lambda i: (0, 0, 0)),
                  pl.BlockSpec((1, D), lambda i: (0, 0))],
        out_specs=pl.BlockSpec((BN, D), lambda i: (i, 0)),
        out_shape=jax.ShapeDtypeStruct((N, D), f32),
    )(p128, denp, b)


def _gin1_body(h_ref, ph_ref, x_ref, px_ref, w1h_ref, w1x_ref, b1_ref,
               z_ref, bs_ref, bq_ref):
    hh = h_ref[...] + ph_ref[0] + ph_ref[1]
    xx = x_ref[...] + px_ref[0] + px_ref[1]
    z = (jnp.dot(hh, w1h_ref[...], preferred_element_type=f32)
         + jnp.dot(xx, w1x_ref[...], preferred_element_type=f32) + b1_ref[...])
    z_ref[...] = z
    bs_ref[...] = jnp.sum(z, axis=0).reshape(1, 1, D)
    bq_ref[...] = jnp.sum(z * z, axis=0).reshape(1, 1, D)


def _gin1(h, ph, x, px, W1h, W1x, b1):
    full = lambda shp: pl.BlockSpec(shp, lambda i: tuple(0 for _ in shp))
    return pl.pallas_call(
        _gin1_body,
        grid=(NB,),
        in_specs=[pl.BlockSpec((BN, D), lambda i: (i, 0)),
                  pl.BlockSpec((NC, BN, D), lambda i: (0, i, 0)),
                  pl.BlockSpec((BN, D), lambda i: (i, 0)),
                  pl.BlockSpec((NC, BN, D), lambda i: (0, i, 0)),
                  full((D, D)), full((D, D)), full((1, D))],
        out_specs=(pl.BlockSpec((BN, D), lambda i: (i, 0)),
                   pl.BlockSpec((1, 1, D), lambda i: (i, 0, 0)),
                   pl.BlockSpec((1, 1, D), lambda i: (i, 0, 0))),
        out_shape=(jax.ShapeDtypeStruct((N, D), f32),
                   jax.ShapeDtypeStruct((NB, 1, D), f32),
                   jax.ShapeDtypeStruct((NB, 1, D), f32)),
    )(h, ph, x, px, W1h, W1x, b1)


def _gin2_body(z_ref, bs_ref, bq_ref, g_ref, be_ref, w2_ref, b2_ref, o_ref):
    mu = jnp.sum(bs_ref[...][:, 0, :], axis=0) * (1.0 / N)
    msq = jnp.sum(bq_ref[...][:, 0, :], axis=0) * (1.0 / N)
    var = msq - mu * mu
    inv = lax.rsqrt(var + 1e-5)
    z = (z_ref[...] - mu) * (inv * g_ref[...]) + be_ref[...]
    z = jnp.maximum(z, 0.0)
    o = jnp.dot(z, w2_ref[...], preferred_element_type=f32) + b2_ref[...]
    o_ref[...] = jnp.maximum(o, 0.0)


def _gin2(z, bs, bq, g, be, W2, b2):
    full = lambda shp: pl.BlockSpec(shp, lambda i: tuple(0 for _ in shp))
    return pl.pallas_call(
        _gin2_body,
        grid=(NB,),
        in_specs=[pl.BlockSpec((BN, D), lambda i: (i, 0)),
                  full((NB, 1, D)), full((NB, 1, D)),
                  full((1, D)), full((1, D)), full((D, D)), full((1, D))],
        out_specs=pl.BlockSpec((BN, D), lambda i: (i, 0)),
        out_shape=jax.ShapeDtypeStruct((N, D), f32),
    )(z, bs, bq, g, be, W2, b2)


def _sc_mesh():
    return plsc.VectorSubcoreMesh(core_axis_name="c", subcore_axis_name="s",
                                  num_cores=NC, num_subcores=NS)


def _gat_fused(feat, elt, ert, src3, dst3, didx3, grp3, b16, zer128):
    def bcast(v, h):
        return jnp.broadcast_to(lax.slice(v, (h,), (h + 1,)), (16,))

    @functools.partial(
        pl.kernel,
        out_type=jax.ShapeDtypeStruct((NC, NA, D), f32),
        mesh=_sc_mesh(),
        scratch_types=[pltpu.VMEM((EB,), jnp.int32),
                       pltpu.VMEM((EB,), jnp.int32),
                       pltpu.VMEM((EB,), jnp.int32),
                       pltpu.VMEM((EB, D), f32),
                       pltpu.VMEM((EB, D), f32),
                       pltpu.VMEM((EB, D), f32),
                       pltpu.VMEM((EB,), f32),
                       pltpu.VMEM((16,), f32),
                       pltpu.VMEM_SHARED((NA, D), f32),
                       pltpu.SemaphoreType.DMA, pltpu.SemaphoreType.DMA,
                       pltpu.SemaphoreType.DMA],
    )
    def k(feat_hbm, el_hbm, er_hbm, src_hbm, dst_hbm, didx_hbm, grp_hbm,
          bnd_hbm, z128_hbm, msg_hbm,
          src_v, dst_v, didx_v, F_v, EL_v, ER_v, gf_v, bnd_v,
          acc_sh, sem1, sem2, sem3):
        c = lax.axis_index("c")
        s = lax.axis_index("s")
        wid = s * NC + c
        r0 = s * RPA

        pltpu.sync_copy(z128_hbm.at[pl.ds(0, RPA)], acc_sh.at[pl.ds(r0, RPA)])
        pltpu.sync_copy(bnd_hbm, bnd_v)
        plsc.subcore_barrier()
        bnd = bnd_v[...]

        def blk(j, carry):
            pltpu.sync_copy(src_hbm.at[wid, j], src_v)
            pltpu.sync_copy(dst_hbm.at[wid, j], dst_v)
            pltpu.sync_copy(didx_hbm.at[wid, j], didx_v)
            pltpu.sync_copy(grp_hbm.at[wid, j], gf_v)
            cp1 = pltpu.async_copy(feat_hbm.at[src_v], F_v, sem1)
            cp2 = pltpu.async_copy(el_hbm.at[src_v], EL_v, sem2)
            cp3 = pltpu.async_copy(er_hbm.at[dst_v], ER_v, sem3)

            cp1.wait()
            cp2.wait()
            cp3.wait()

            for q in range(EB // 16):
                gf = gf_v[pl.ds(q * 16, 16)]
                for t in range(16):
                    e = q * 16 + t
                    sv = EL_v[e, pl.ds(0, 16)] + ER_v[e, pl.ds(0, 16)]
                    sv = jnp.maximum(sv, 0.2 * sv)
                    ee = jnp.exp(sv - bnd)
                    g16 = jnp.broadcast_to(lax.slice(gf, (t,), (t + 1,)), (16,))
                    for g in range(8):
                        dif = g16 - float(g)
                        m = jnp.maximum(1.0 - jnp.maximum(dif, -dif), 0.0)
                        EL_v[e, pl.ds(g * 16, 16)] = ee * m
                    for h in range(H):
                        bc = bcast(ee, h)
                        for qq in range(2):
                            off = h * Fh + qq * 16
                            F_v[e, pl.ds(off, 16)] = F_v[e, pl.ds(off, 16)] * bc

            pltpu.sync_copy(F_v, acc_sh.at[dst_v], add=True)
            pltpu.sync_copy(EL_v, acc_sh.at[didx_v], add=True)
            return carry

        lax.fori_loop(0, NBLK, blk, 0)
        plsc.subcore_barrier()

        pltpu.sync_copy(acc_sh.at[pl.ds(r0, RPA)], msg_hbm.at[c, pl.ds(r0, RPA)])

    return k(feat, elt, ert, src3, dst3, didx3, grp3, b16, zer128)


def _gsa(table, src3, dst3, zer128):
    @functools.partial(
        pl.kernel,
        out_type=jax.ShapeDtypeStruct((NC, N, D), f32),
        mesh=_sc_mesh(),
        scratch_types=[pltpu.VMEM((EB,), jnp.int32), pltpu.VMEM((EB,), jnp.int32),
                       pltpu.VMEM((EB, D), f32), pltpu.VMEM_SHARED((N, D), f32),
                       pltpu.SemaphoreType.DMA],
    )
    def k(tab_hbm, src_hbm, dst_hbm, z128_hbm, p_hbm,
          src_v, dst_v, rows_v, acc_sh, sem):
        c = lax.axis_index("c")
        s = lax.axis_index("s")
        wid = s * NC + c
        r0 = s * RPS

        @pl.when(s < NS - 1)
        def _():
            pltpu.sync_copy(z128_hbm.at[pl.ds(0, RPS)], acc_sh.at[pl.ds(r0, RPS)])

        @pl.when(s == NS - 1)
        def _():
            pltpu.sync_copy(z128_hbm.at[pl.ds(0, RLAST)], acc_sh.at[pl.ds(r0, RLAST)])

        plsc.subcore_barrier()

        def body(j, carry):
            pltpu.sync_copy(src_hbm.at[wid, j], src_v)
            pltpu.sync_copy(dst_hbm.at[wid, j], dst_v)
            pltpu.async_copy(tab_hbm.at[src_v], rows_v, sem).wait()
            pltpu.sync_copy(rows_v, acc_sh.at[dst_v], add=True)
            return carry

        lax.fori_loop(0, NBLK, body, 0)
        plsc.subcore_barrier()

        @pl.when(s < NS - 1)
        def _():
            pltpu.sync_copy(acc_sh.at[pl.ds(r0, RPS)], p_hbm.at[c, pl.ds(r0, RPS)])

        @pl.when(s == NS - 1)
        def _():
            pltpu.sync_copy(acc_sh.at[pl.ds(r0, RLAST)], p_hbm.at[c, pl.ds(r0, RLAST)])

    return k(table, src3, dst3, zer128)


def _bound16(bmA, bmB):
    c = jnp.max(bmA[:, 0, :], axis=0) + jnp.max(bmB[:, 0, :], axis=0)
    return jnp.where(c >= 0, c, 0.2 * c)


def kernel(x, edge_index_0, edge_index_1,
           gat_W_0, gat_al_0, gat_ar_0, gat_b_0,
           gin_W1_0, gin_b1_0, gin_g_0, gin_be_0, gin_W2_0, gin_b2_0,
           gat_W_1, gat_al_1, gat_ar_1, gat_b_1,
           gin_W1_1, gin_b1_1, gin_g_1, gin_be_1, gin_W2_1, gin_b2_1):
    al0 = gat_al_0.reshape(1, D)
    ar0 = gat_ar_0.reshape(1, D)
    al1 = gat_al_1.reshape(1, D)
    ar1 = gat_ar_1.reshape(1, D)
    zer128 = jnp.zeros((RLA, D), f32)

    (feat0, el0t, er0t, bmA0, bmB0,
     feat1, el1t, er1t, bmA1, bmB1) = _prep(
        x, gat_W_0, al0, ar0, gat_W_1, al1, ar1)

    cfgs = [
        (edge_index_0, feat0, el0t, er0t, _bound16(bmA0, bmB0), gat_b_0,
         gin_W1_0, gin_b1_0, gin_g_0, gin_be_0, gin_W2_0, gin_b2_0),
        (edge_index_1, feat1, el1t, er1t, _bound16(bmA1, bmB1), gat_b_1,
         gin_W1_1, gin_b1_1, gin_g_1, gin_be_1, gin_W2_1, gin_b2_1),
    ]
    outs = []
    for (ei, feat, elt, ert, b16, gb, W1, b1, g, be, W2, b2) in cfgs:
        src3 = ei[0].reshape(NW, NBLK, EB)
        dst3 = ei[1].reshape(NW, NBLK, EB)
        aggx_p = _gsa(x, src3, dst3, zer128)
        didx3 = (dst3 // 8 + N).astype(jnp.int32)
        grp3 = (dst3 % 8).astype(f32)
        msg_p = _gat_fused(feat, elt, ert, src3, dst3, didx3, grp3,
                           b16, zer128)
        den_p = lax.slice(msg_p, (0, N, 0), (NC, N + DR, D))
        h = _norm(msg_p, den_p, gb.reshape(1, D))
        aggh_p = _gsa(h, src3, dst3, zer128)
        z, bs, bq = _gin1(h, aggh_p, x, aggx_p, W1[:D], W1[D:],
                          b1.reshape(1, D))
        o = _gin2(z, bs, bq, g.reshape(1, D), be.reshape(1, D),
                  W2, b2.reshape(1, D))
        outs.append(o)
    return jnp.concatenate(outs, axis=1)


# speedup vs baseline: 26.0708x; 1.0624x over previous
import functools

import jax
import jax.numpy as jnp
from jax import lax
from jax.experimental import pallas as pl
from jax.experimental.pallas import tpu as pltpu
from jax.experimental.pallas import tpu_sc as plsc

N = 10000
E = 320000
D = 128
H = 4
Fh = 32

NC = 2
NS = 16
NW = NC * NS
EPW = E // NW
EB = 80
NBLK = EPW // EB
RPS = 624
RLAST = N - 15 * RPS
DR = N // 8
NA = 11264
RPA = 704
RLA = NA - 15 * RPA

NB = 10
BN = N // NB

f32 = jnp.float32


def _gmask():
    rr = lax.broadcasted_iota(jnp.int32, (D, 16), 0)
    cc = lax.broadcasted_iota(jnp.int32, (D, 16), 1)
    return ((rr // Fh) == cc).astype(f32)


def _gmask128():
    rr = lax.broadcasted_iota(jnp.int32, (D, D), 0)
    cc = lax.broadcasted_iota(jnp.int32, (D, D), 1)
    return (((rr // Fh) == cc) & (cc < 16)).astype(f32)


def _pmask():
    rr = lax.broadcasted_iota(jnp.int32, (16, D), 0)
    cc = lax.broadcasted_iota(jnp.int32, (16, D), 1)
    return (((cc // Fh) == rr) & (rr < H)).astype(f32)


def _prep_body(x_ref, w0, al0, ar0, w1, al1, ar1,
               feat0, el0, er0, bmA0, bmB0,
               feat1, el1, er1, bmA1, bmB1):
    x = x_ref[...]
    G = _gmask()

    G128 = _gmask128()

    def one(w, alf, arf, featref, elref, erref, bmAref, bmBref):
        feat = jnp.dot(x, w[...], preferred_element_type=f32)
        featref[...] = feat
        el16 = jnp.dot(feat * alf[...], G, preferred_element_type=f32)
        er16 = jnp.dot(feat * arf[...], G, preferred_element_type=f32)
        elref[...] = jnp.dot(feat * alf[...], G128, preferred_element_type=f32)
        erref[...] = jnp.dot(feat * arf[...], G128, preferred_element_type=f32)
        bmAref[...] = jnp.max(el16, axis=0).reshape(1, 1, 16)
        bmBref[...] = jnp.max(er16, axis=0).reshape(1, 1, 16)

    one(w0, al0, ar0, feat0, el0, er0, bmA0, bmB0)
    one(w1, al1, ar1, feat1, el1, er1, bmA1, bmB1)


def _prep(x, W0, al0, ar0, W1, al1, ar1):
    full = lambda shp: pl.BlockSpec(shp, lambda i: tuple(0 for _ in shp))
    outs = (jax.ShapeDtypeStruct((N, D), f32),
            jax.ShapeDtypeStruct((N, D), f32),
            jax.ShapeDtypeStruct((N, D), f32),
            jax.ShapeDtypeStruct((NB, 1, 16), f32),
            jax.ShapeDtypeStruct((NB, 1, 16), f32))
    return pl.pallas_call(
        _prep_body,
        grid=(NB,),
        in_specs=[pl.BlockSpec((BN, D), lambda i: (i, 0)),
                  full((D, D)), full((1, D)), full((1, D)),
                  full((D, D)), full((1, D)), full((1, D))],
        out_specs=(pl.BlockSpec((BN, D), lambda i: (i, 0)),
                   pl.BlockSpec((BN, D), lambda i: (i, 0)),
                   pl.BlockSpec((BN, D), lambda i: (i, 0)),
                   pl.BlockSpec((1, 1, 16), lambda i: (i, 0, 0)),
                   pl.BlockSpec((1, 1, 16), lambda i: (i, 0, 0))) * 2,
        out_shape=outs * 2,
    )(x, W0, al0, ar0, W1, al1, ar1)


def _norm_body(p128_ref, pden_ref, b_ref, h_ref):
    i = pl.program_id(0)
    acc = p128_ref[0] + p128_ref[1]
    denp = pden_ref[0] + pden_ref[1]
    nn = lax.broadcasted_iota(jnp.int32, (BN, DR), 0)
    rc = lax.broadcasted_iota(jnp.int32, (BN, DR), 1)
    P = (rc == i * (BN // 8) + nn // 8).astype(f32)
    tmp = jnp.dot(P, denp, preferred_element_type=f32)
    ll = lax.broadcasted_iota(jnp.int32, (D, D), 0)
    dd = lax.broadcasted_iota(jnp.int32, (D, D), 1)
    nv = lax.broadcasted_iota(jnp.int32, (BN, 1), 0)
    denx = jnp.zeros_like(acc)
    for g in range(8):
        Mg = (ll == g * 16 + dd // Fh).astype(f32)
        mg = ((nv % 8) == g).astype(f32)
        denx = denx + mg * jnp.dot(tmp, Mg, preferred_element_type=f32)
    out = acc / jnp.where(denx > 0, denx, 1.0) + b_ref[...]
    h_ref[...] = jnp.maximum(out, 0.0)


def _norm(p128, denp, b):
    return pl.pallas_call(
        _norm_body,
        grid=(NB,),
        in_specs=[pl.BlockSpec((NC, BN, D), lambda i: (0, i, 0)),
                  pl.BlockSpec((NC, DR, D), lambda i: (0, 0, 0)),
                  pl.BlockSpec((1, D), lambda i: (0, 0))],
        out_specs=pl.BlockSpec((BN, D), lambda i: (i, 0)),
        out_shape=jax.ShapeDtypeStruct((N, D), f32),
    )(p128, denp, b)


def _gin1_body(h_ref, ph_ref, x_ref, px_ref, w1h_ref, w1x_ref, b1_ref,
               z_ref, bs_ref, bq_ref):
    hh = h_ref[...] + ph_ref[0] + ph_ref[1]
    xx = x_ref[...] + px_ref[0] + px_ref[1]
    z = (jnp.dot(hh, w1h_ref[...], preferred_element_type=f32)
         + jnp.dot(xx, w1x_ref[...], preferred_element_type=f32) + b1_ref[...])
    z_ref[...] = z
    bs_ref[...] = jnp.sum(z, axis=0).reshape(1, 1, D)
    bq_ref[...] = jnp.sum(z * z, axis=0).reshape(1, 1, D)


def _gin1(h, ph, x, px, W1h, W1x, b1):
    full = lambda shp: pl.BlockSpec(shp, lambda i: tuple(0 for _ in shp))
    return pl.pallas_call(
        _gin1_body,
        grid=(NB,),
        in_specs=[pl.BlockSpec((BN, D), lambda i: (i, 0)),
                  pl.BlockSpec((NC, BN, D), lambda i: (0, i, 0)),
                  pl.BlockSpec((BN, D), lambda i: (i, 0)),
                  pl.BlockSpec((NC, BN, D), lambda i: (0, i, 0)),
                  full((D, D)), full((D, D)), full((1, D))],
        out_specs=(pl.BlockSpec((BN, D), lambda i: (i, 0)),
                   pl.BlockSpec((1, 1, D), lambda i: (i, 0, 0)),
                   pl.BlockSpec((1, 1, D), lambda i: (i, 0, 0))),
        out_shape=(jax.ShapeDtypeStruct((N, D), f32),
                   jax.ShapeDtypeStruct((NB, 1, D), f32),
                   jax.ShapeDtypeStruct((NB, 1, D), f32)),
    )(h, ph, x, px, W1h, W1x, b1)


def _gin2_body(z_ref, bs_ref, bq_ref, g_ref, be_ref, w2_ref, b2_ref, o_ref):
    mu = jnp.sum(bs_ref[...][:, 0, :], axis=0) * (1.0 / N)
    msq = jnp.sum(bq_ref[...][:, 0, :], axis=0) * (1.0 / N)
    var = msq - mu * mu
    inv = lax.rsqrt(var + 1e-5)
    z = (z_ref[...] - mu) * (inv * g_ref[...]) + be_ref[...]
    z = jnp.maximum(z, 0.0)
    o = jnp.dot(z, w2_ref[...], preferred_element_type=f32) + b2_ref[...]
    o_ref[...] = jnp.maximum(o, 0.0)


def _gin2(z, bs, bq, g, be, W2, b2):
    full = lambda shp: pl.BlockSpec(shp, lambda i: tuple(0 for _ in shp))
    return pl.pallas_call(
        _gin2_body,
        grid=(NB,),
        in_specs=[pl.BlockSpec((BN, D), lambda i: (i, 0)),
                  full((NB, 1, D)), full((NB, 1, D)),
                  full((1, D)), full((1, D)), full((D, D)), full((1, D))],
        out_specs=pl.BlockSpec((BN, D), lambda i: (i, 0)),
        out_shape=jax.ShapeDtypeStruct((N, D), f32),
    )(z, bs, bq, g, be, W2, b2)


def _sc_mesh():
    return plsc.VectorSubcoreMesh(core_axis_name="c", subcore_axis_name="s",
                                  num_cores=NC, num_subcores=NS)


def _gat_fused(feat, elt, ert, idx4, grp3, b16, zer128):
    def bcast(v, h):
        return jnp.broadcast_to(lax.slice(v, (h,), (h + 1,)), (16,))

    @functools.partial(
        pl.kernel,
        out_type=jax.ShapeDtypeStruct((NC, NA, D), f32),
        mesh=_sc_mesh(),
        scratch_types=[pltpu.VMEM((2, EB), jnp.int32),
                       pltpu.VMEM((EB,), jnp.int32),
                       pltpu.VMEM((EB, D), f32),
                       pltpu.VMEM((EB, D), f32),
                       pltpu.VMEM((EB, D), f32),
                       pltpu.VMEM((EB,), f32),
                       pltpu.VMEM((16,), f32),
                       pltpu.VMEM_SHARED((NA, D), f32),
                       pltpu.SemaphoreType.DMA, pltpu.SemaphoreType.DMA,
                       pltpu.SemaphoreType.DMA],
    )
    def k(feat_hbm, el_hbm, er_hbm, idx_hbm, grp_hbm,
          bnd_hbm, z128_hbm, msg_hbm,
          IDX_v, didx_v, F_v, EL_v, ER_v, gf_v, bnd_v,
          acc_sh, sem1, sem2, sem3):
        c = lax.axis_index("c")
        s = lax.axis_index("s")
        wid = s * NC + c
        r0 = s * RPA

        pltpu.sync_copy(z128_hbm.at[pl.ds(0, RPA)], acc_sh.at[pl.ds(r0, RPA)])
        pltpu.sync_copy(bnd_hbm, bnd_v)
        plsc.subcore_barrier()
        bnd = bnd_v[...]

        def blk(j, carry):
            pltpu.sync_copy(idx_hbm.at[wid, j], IDX_v)
            pltpu.sync_copy(grp_hbm.at[wid, j], gf_v)
            cp1 = pltpu.async_copy(feat_hbm.at[IDX_v.at[0]], F_v, sem1)
            cp2 = pltpu.async_copy(el_hbm.at[IDX_v.at[0]], EL_v, sem2)
            cp3 = pltpu.async_copy(er_hbm.at[IDX_v.at[1]], ER_v, sem3)

            for q in range(EB // 16):
                dv = IDX_v[1, pl.ds(q * 16, 16)]
                didx_v[pl.ds(q * 16, 16)] = (dv >> 3) + N

            cp1.wait()
            cp2.wait()
            cp3.wait()

            for q in range(EB // 16):
                gf = gf_v[pl.ds(q * 16, 16)]
                for t in range(16):
                    e = q * 16 + t
                    sv = EL_v[e, pl.ds(0, 16)] + ER_v[e, pl.ds(0, 16)]
                    sv = jnp.maximum(sv, 0.2 * sv)
                    ee = jnp.exp(sv - bnd)
                    g16 = jnp.broadcast_to(lax.slice(gf, (t,), (t + 1,)), (16,))
                    for g in range(8):
                        dif = g16 - float(g)
                        m = jnp.maximum(1.0 - jnp.maximum(dif, -dif), 0.0)
                        EL_v[e, pl.ds(g * 16, 16)] = ee * m
                    for h in range(H):
                        bc = bcast(ee, h)
                        for qq in range(2):
                            off = h * Fh + qq * 16
                            F_v[e, pl.ds(off, 16)] = F_v[e, pl.ds(off, 16)] * bc

            pltpu.sync_copy(F_v, acc_sh.at[IDX_v.at[1]], add=True)
            pltpu.sync_copy(EL_v, acc_sh.at[didx_v], add=True)
            return carry

        lax.fori_loop(0, NBLK, blk, 0)
        plsc.subcore_barrier()

        pltpu.sync_copy(acc_sh.at[pl.ds(r0, RPA)], msg_hbm.at[c, pl.ds(r0, RPA)])

    return k(feat, elt, ert, idx4, grp3, b16, zer128)


def _gsa(table, src3, dst3, zer128):
    @functools.partial(
        pl.kernel,
        out_type=jax.ShapeDtypeStruct((NC, N, D), f32),
        mesh=_sc_mesh(),
        scratch_types=[pltpu.VMEM((EB,), jnp.int32), pltpu.VMEM((EB,), jnp.int32),
                       pltpu.VMEM((EB, D), f32), pltpu.VMEM_SHARED((N, D), f32),
                       pltpu.SemaphoreType.DMA],
    )
    def k(tab_hbm, src_hbm, dst_hbm, z128_hbm, p_hbm,
          src_v, dst_v, rows_v, acc_sh, sem):
        c = lax.axis_index("c")
        s = lax.axis_index("s")
        wid = s * NC + c
        r0 = s * RPS

        @pl.when(s < NS - 1)
        def _():
            pltpu.sync_copy(z128_hbm.at[pl.ds(0, RPS)], acc_sh.at[pl.ds(r0, RPS)])

        @pl.when(s == NS - 1)
        def _():
            pltpu.sync_copy(z128_hbm.at[pl.ds(0, RLAST)], acc_sh.at[pl.ds(r0, RLAST)])

        plsc.subcore_barrier()

        def body(j, carry):
            pltpu.sync_copy(src_hbm.at[wid, j], src_v)
            pltpu.sync_copy(dst_hbm.at[wid, j], dst_v)
            pltpu.async_copy(tab_hbm.at[src_v], rows_v, sem).wait()
            pltpu.sync_copy(rows_v, acc_sh.at[dst_v], add=True)
            return carry

        lax.fori_loop(0, NBLK, body, 0)
        plsc.subcore_barrier()

        @pl.when(s < NS - 1)
        def _():
            pltpu.sync_copy(acc_sh.at[pl.ds(r0, RPS)], p_hbm.at[c, pl.ds(r0, RPS)])

        @pl.when(s == NS - 1)
        def _():
            pltpu.sync_copy(acc_sh.at[pl.ds(r0, RLAST)], p_hbm.at[c, pl.ds(r0, RLAST)])

    return k(table, src3, dst3, zer128)


def _bound16(bmA, bmB):
    c = jnp.max(bmA[:, 0, :], axis=0) + jnp.max(bmB[:, 0, :], axis=0)
    return jnp.where(c >= 0, c, 0.2 * c)


def kernel(x, edge_index_0, edge_index_1,
           gat_W_0, gat_al_0, gat_ar_0, gat_b_0,
           gin_W1_0, gin_b1_0, gin_g_0, gin_be_0, gin_W2_0, gin_b2_0,
           gat_W_1, gat_al_1, gat_ar_1, gat_b_1,
           gin_W1_1, gin_b1_1, gin_g_1, gin_be_1, gin_W2_1, gin_b2_1):
    al0 = gat_al_0.reshape(1, D)
    ar0 = gat_ar_0.reshape(1, D)
    al1 = gat_al_1.reshape(1, D)
    ar1 = gat_ar_1.reshape(1, D)
    zer128 = jnp.zeros((RLA, D), f32)

    (feat0, el0t, er0t, bmA0, bmB0,
     feat1, el1t, er1t, bmA1, bmB1) = _prep(
        x, gat_W_0, al0, ar0, gat_W_1, al1, ar1)

    cfgs = [
        (edge_index_0, feat0, el0t, er0t, _bound16(bmA0, bmB0), gat_b_0,
         gin_W1_0, gin_b1_0, gin_g_0, gin_be_0, gin_W2_0, gin_b2_0),
        (edge_index_1, feat1, el1t, er1t, _bound16(bmA1, bmB1), gat_b_1,
         gin_W1_1, gin_b1_1, gin_g_1, gin_be_1, gin_W2_1, gin_b2_1),
    ]
    outs = []
    for (ei, feat, elt, ert, b16, gb, W1, b1, g, be, W2, b2) in cfgs:
        src3 = ei[0].reshape(NW, NBLK, EB)
        dst3 = ei[1].reshape(NW, NBLK, EB)
        aggx_p = _gsa(x, src3, dst3, zer128)
        idx4 = jnp.stack([src3, dst3], axis=2)
        grp3 = (dst3 % 8).astype(f32)
        msg_p = _gat_fused(feat, elt, ert, idx4, grp3, b16, zer128)
        den_p = lax.slice(msg_p, (0, N, 0), (NC, N + DR, D))
        h = _norm(msg_p, den_p, gb.reshape(1, D))
        aggh_p = _gsa(h, src3, dst3, zer128)
        z, bs, bq = _gin1(h, aggh_p, x, aggx_p, W1[:D], W1[D:],
                          b1.reshape(1, D))
        o = _gin2(z, bs, bq, g.reshape(1, D), be.reshape(1, D),
                  W2, b2.reshape(1, D))
        outs.append(o)
    return jnp.concatenate(outs, axis=1)
